# Initial kernel scaffold; baseline (speedup 1.0000x reference)
import jax
import jax.numpy as jnp
from jax.experimental import pallas as pl


def kernel(x_foundation, x_expression, Wl0, bl0, Wr0, Wl1, bl1, Wr1, Wf, bf, g1, be1, Wfu, bfu, g2, be2, Wfl, bfl, Wfr, ppi_edge_index):
    raise NotImplementedError("write your pallas kernel here")


# speedup vs baseline: 2.9050x; 2.9050x over previous
import functools

import jax
import jax.numpy as jnp
from jax import lax
from jax.experimental import pallas as pl
from jax.experimental.pallas import tpu as pltpu
from jax.experimental.pallas import tpu_sc as plsc

N = 10000
E = 320000
D = 128
D_FM = 512

NC = 2
NS = 16
NW = NC * NS
FPW = D // NW
CH = 10000
NCHUNK = E // CH


def _make_segsum(with_counts: bool):
    mesh = plsc.VectorSubcoreMesh(
        core_axis_name="c", subcore_axis_name="s",
        num_cores=NC, num_subcores=NS)

    out_type = [jax.ShapeDtypeStruct((D * N,), jnp.float32)]
    scratch = [
        pltpu.VMEM((FPW * N,), jnp.float32),
        pltpu.VMEM((FPW * N,), jnp.float32),
        pltpu.VMEM((CH,), jnp.int32),
        pltpu.VMEM((CH,), jnp.int32),
    ]
    if with_counts:
        out_type.append(jax.ShapeDtypeStruct((N,), jnp.float32))
        scratch.append(pltpu.VMEM((N,), jnp.float32))

    def body(xT, src, dst, *rest):
        if with_counts:
            outT, cnt_out, table_v, acc_v, src_buf, dst_buf, cnt_v = rest
        else:
            outT, table_v, acc_v, src_buf, dst_buf = rest
            cnt_v = None

        wid = lax.axis_index("s") * NC + lax.axis_index("c")
        r0 = pl.multiple_of(wid * (FPW * N), 8)
        pltpu.sync_copy(xT.at[pl.ds(r0, FPW * N)], table_v)

        zeros = jnp.zeros((16,), jnp.float32)

        def zbody(j, carry):
            o = pl.multiple_of(j * 16, 16)
            for f in range(FPW):
                acc_v[pl.ds(o + f * N, 16)] = zeros
            if with_counts:
                cnt_v[pl.ds(o, 16)] = zeros
            return carry
        lax.fori_loop(0, N // 16, zbody, 0)

        ones = jnp.ones((16,), jnp.float32)

        def chunk(c, carry):
            off = pl.multiple_of(c * CH, 8)
            pltpu.sync_copy(src.at[pl.ds(off, CH)], src_buf)
            pltpu.sync_copy(dst.at[pl.ds(off, CH)], dst_buf)

            def grp(j, carry2):
                o = pl.multiple_of(j * 16, 16)
                s16 = src_buf[pl.ds(o, 16)]
                d16 = dst_buf[pl.ds(o, 16)]
                for f in range(FPW):
                    v = plsc.load_gather(table_v, [s16 + (f * N)])
                    plsc.addupdate_scatter(acc_v, [d16 + (f * N)], v)
                if with_counts:
                    plsc.addupdate_scatter(cnt_v, [d16], ones)
                return carry2
            lax.fori_loop(0, CH // 16, grp, 0)
            return carry
        lax.fori_loop(0, NCHUNK, chunk, 0)

        pltpu.sync_copy(acc_v, outT.at[pl.ds(r0, FPW * N)])
        if with_counts:
            @pl.when(wid == 0)
            def _():
                pltpu.sync_copy(cnt_v, cnt_out)

    return pl.kernel(body, out_type=out_type, mesh=mesh,
                     scratch_types=scratch,
                     compiler_params=pltpu.CompilerParams(
                         needs_layout_passes=False))


@functools.lru_cache(maxsize=None)
def _get_segsum(with_counts: bool):
    return _make_segsum(with_counts)


_DN00 = (((0,), (0,)), ((), ()))
_DN01 = (((0,), (1,)), ((), ()))


def _elu(z):
    return jnp.where(z > 0, z, jnp.exp(jnp.minimum(z, 0.0)) - 1.0)


def _ln0(z, g, b):
    m = jnp.mean(z, axis=0, keepdims=True)
    v = jnp.mean((z - m) * (z - m), axis=0, keepdims=True)
    return (z - m) * lax.rsqrt(v + 1e-5) * g + b


def _tc_found_body(xf_ref, Wf_ref, bf_ref, g_ref, be_ref, out_ref):
    z = lax.dot_general(Wf_ref[...], xf_ref[...], _DN01,
                        preferred_element_type=jnp.float32)
    z = z + bf_ref[...]
    z = _ln0(z, g_ref[...], be_ref[...])
    out_ref[...] = 0.5 * z * (1.0 + lax.erf(z * 0.7071067811865476))


def _tc_sage_body(s_ref, x_ref, cnt_ref, Wl_ref, bl_ref, Wr_ref, out_ref):
    inv = 1.0 / jnp.maximum(cnt_ref[...], 1.0)
    mean = s_ref[...] * inv
    z = lax.dot_general(Wl_ref[...], mean, _DN00,
                        preferred_element_type=jnp.float32)
    z = z + lax.dot_general(Wr_ref[...], x_ref[...], _DN00,
                            preferred_element_type=jnp.float32)
    z = z + bl_ref[...]
    out_ref[...] = _elu(z)


def _tc_mid_body(s_ref, h_ref, f_ref, cnt_ref, Wl_ref, bl_ref, Wr_ref,
                 Wfu_ref, bfu_ref, g_ref, be_ref, out_ref):
    inv = 1.0 / jnp.maximum(cnt_ref[...], 1.0)
    mean = s_ref[...] * inv
    z = lax.dot_general(Wl_ref[...], mean, _DN00,
                        preferred_element_type=jnp.float32)
    z = z + lax.dot_general(Wr_ref[...], h_ref[...], _DN00,
                            preferred_element_type=jnp.float32)
    z = z + bl_ref[...]
    fused = _elu(z) + f_ref[...]
    z2 = lax.dot_general(Wfu_ref[...], fused, _DN00,
                         preferred_element_type=jnp.float32)
    z2 = z2 + bfu_ref[...]
    z2 = _ln0(z2, g_ref[...], be_ref[...])
    out_ref[...] = z2 * jax.nn.sigmoid(z2)


def _tc_final_body(s_ref, x_ref, cnt_ref, Wfl_ref, bfl_ref, Wfr_ref, out_ref):
    inv = 1.0 / jnp.maximum(cnt_ref[...], 1.0)
    mean = s_ref[...] * inv
    z = lax.dot_general(mean, Wfl_ref[...], _DN00,
                        preferred_element_type=jnp.float32)
    z = z + lax.dot_general(x_ref[...], Wfr_ref[...], _DN00,
                            preferred_element_type=jnp.float32)
    z = z + bfl_ref[...]
    out_ref[...] = _elu(z)


def _tc_call(body, n_in, out_shape):
    return pl.pallas_call(body, out_shape=out_shape)


_tc_found = pl.pallas_call(
    _tc_found_body, out_shape=jax.ShapeDtypeStruct((D, N), jnp.float32))
_tc_sage = pl.pallas_call(
    _tc_sage_body, out_shape=jax.ShapeDtypeStruct((D, N), jnp.float32))
_tc_mid = pl.pallas_call(
    _tc_mid_body, out_shape=jax.ShapeDtypeStruct((D, N), jnp.float32))
_tc_final = pl.pallas_call(
    _tc_final_body, out_shape=jax.ShapeDtypeStruct((N, D), jnp.float32))


def kernel(x_foundation, x_expression, Wl0, bl0, Wr0, Wl1, bl1, Wr1,
           Wf, bf, g1, be1, Wfu, bfu, g2, be2, Wfl, bfl, Wfr,
           ppi_edge_index):
    src = ppi_edge_index[0]
    dst = ppi_edge_index[1]
    xeT = x_expression.T

    s0f, cnt = _get_segsum(True)(xeT.reshape(-1), src, dst)
    s0T = s0f.reshape(D, N)
    cnt_r = cnt.reshape(1, N)

    fT = _tc_found(x_foundation, Wf, bf.reshape(D, 1), g1.reshape(D, 1),
                   be1.reshape(D, 1))
    h1T = _tc_sage(s0T, xeT, cnt_r, Wl0, bl0.reshape(D, 1), Wr0)

    (s1f,) = _get_segsum(False)(h1T.reshape(-1), src, dst)
    s1T = s1f.reshape(D, N)
    preT = _tc_mid(s1T, h1T, fT, cnt_r, Wl1, bl1.reshape(D, 1), Wr1,
                   Wfu, bfu.reshape(D, 1), g2.reshape(D, 1), be2.reshape(D, 1))

    (s2f,) = _get_segsum(False)(preT.reshape(-1), src, dst)
    s2T = s2f.reshape(D, N)
    out = _tc_final(s2T, preT, cnt_r, Wfl, bfl.reshape(1, D), Wfr)
    return out


# speedup vs baseline: 3.2987x; 1.1355x over previous
import functools

import jax
import jax.numpy as jnp
from jax import lax
from jax.experimental import pallas as pl
from jax.experimental.pallas import tpu as pltpu
from jax.experimental.pallas import tpu_sc as plsc

N = 10000
E = 320000
D = 128
D_FM = 512

NC = 2
NS = 16
NW = NC * NS
FPW = D // NW
CH = 6400
NCHUNK = E // CH
UNROLL = 8


def _make_segsum(with_counts: bool):
    mesh = plsc.VectorSubcoreMesh(
        core_axis_name="c", subcore_axis_name="s",
        num_cores=NC, num_subcores=NS)

    out_type = [jax.ShapeDtypeStruct((D * N,), jnp.float32)]
    scratch = [
        pltpu.VMEM((FPW * N,), jnp.float32),
        pltpu.VMEM((FPW * N,), jnp.float32),
        pltpu.VMEM((CH,), jnp.int32),
        pltpu.VMEM((CH,), jnp.int32),
        pltpu.VMEM((CH,), jnp.int32),
        pltpu.VMEM((CH,), jnp.int32),
        pltpu.SemaphoreType.DMA,
        pltpu.SemaphoreType.DMA,
        pltpu.SemaphoreType.DMA,
        pltpu.SemaphoreType.DMA,
    ]
    if with_counts:
        out_type.append(jax.ShapeDtypeStruct((N,), jnp.float32))
        scratch.append(pltpu.VMEM((N,), jnp.float32))

    def body(xT, src, dst, *rest):
        if with_counts:
            (outT, cnt_out, table_v, acc_v, sb0, sb1, db0, db1,
             ss0, ss1, ds0, ds1, cnt_v) = rest
        else:
            (outT, table_v, acc_v, sb0, sb1, db0, db1,
             ss0, ss1, ds0, ds1) = rest
            cnt_v = None
        slots = ((sb0, db0, ss0, ds0), (sb1, db1, ss1, ds1))

        def fill(c, slot):
            sb, db, ssem, dsem = slots[slot]
            off = pl.multiple_of(c * CH, 8)
            pltpu.async_copy(src.at[pl.ds(off, CH)], sb, ssem)
            pltpu.async_copy(dst.at[pl.ds(off, CH)], db, dsem)

        def drain(c, slot):
            sb, db, ssem, dsem = slots[slot]
            off = pl.multiple_of(c * CH, 8)
            pltpu.make_async_copy(src.at[pl.ds(off, CH)], sb, ssem).wait()
            pltpu.make_async_copy(dst.at[pl.ds(off, CH)], db, dsem).wait()

        fill(0, 0)
        fill(1, 1)

        wid = lax.axis_index("s") * NC + lax.axis_index("c")
        r0 = pl.multiple_of(wid * (FPW * N), 8)
        pltpu.sync_copy(xT.at[pl.ds(r0, FPW * N)], table_v)

        zeros = jnp.zeros((16,), jnp.float32)

        def zbody(j, carry):
            o = pl.multiple_of(j * 80, 16)
            for u in range(5):
                for f in range(FPW):
                    acc_v[pl.ds(o + u * 16 + f * N, 16)] = zeros
                if with_counts:
                    cnt_v[pl.ds(o + u * 16, 16)] = zeros
            return carry
        lax.fori_loop(0, N // 80, zbody, 0)

        ones = jnp.ones((16,), jnp.float32)

        def process(sb, db):
            def grp(j, carry2):
                base = pl.multiple_of(j * (16 * UNROLL), 16)
                for u in range(UNROLL):
                    o = pl.multiple_of(base + u * 16, 16)
                    s16 = sb[pl.ds(o, 16)]
                    d16 = db[pl.ds(o, 16)]
                    for f in range(FPW):
                        v = plsc.load_gather(table_v, [s16 + (f * N)])
                        plsc.addupdate_scatter(acc_v, [d16 + (f * N)], v)
                    if with_counts:
                        plsc.addupdate_scatter(cnt_v, [d16], ones)
                return carry2
            lax.fori_loop(0, CH // (16 * UNROLL), grp, 0)

        def pair(p, carry):
            for slot in range(2):
                c = p * 2 + slot
                drain(c, slot)
                process(*slots[slot][:2])

                @pl.when(c + 2 < NCHUNK)
                def _():
                    fill(c + 2, slot)
            return carry
        lax.fori_loop(0, NCHUNK // 2, pair, 0)

        pltpu.sync_copy(acc_v, outT.at[pl.ds(r0, FPW * N)])
        if with_counts:
            @pl.when(wid == 0)
            def _():
                pltpu.sync_copy(cnt_v, cnt_out)

    return pl.kernel(body, out_type=out_type, mesh=mesh,
                     scratch_types=scratch,
                     compiler_params=pltpu.CompilerParams(
                         needs_layout_passes=False))


@functools.lru_cache(maxsize=None)
def _get_segsum(with_counts: bool):
    return _make_segsum(with_counts)


_DN00 = (((0,), (0,)), ((), ()))
_DN01 = (((0,), (1,)), ((), ()))


def _elu(z):
    return jnp.where(z > 0, z, jnp.exp(jnp.minimum(z, 0.0)) - 1.0)


def _ln0(z, g, b):
    m = jnp.mean(z, axis=0, keepdims=True)
    v = jnp.mean((z - m) * (z - m), axis=0, keepdims=True)
    return (z - m) * lax.rsqrt(v + 1e-5) * g + b


def _tc_found_body(xf_ref, Wf_ref, bf_ref, g_ref, be_ref, out_ref):
    z = lax.dot_general(Wf_ref[...], xf_ref[...], _DN01,
                        preferred_element_type=jnp.float32)
    z = z + bf_ref[...]
    z = _ln0(z, g_ref[...], be_ref[...])
    out_ref[...] = 0.5 * z * (1.0 + lax.erf(z * 0.7071067811865476))


def _tc_sage_body(s_ref, x_ref, cnt_ref, Wl_ref, bl_ref, Wr_ref, out_ref):
    inv = 1.0 / jnp.maximum(cnt_ref[...], 1.0)
    mean = s_ref[...] * inv
    z = lax.dot_general(Wl_ref[...], mean, _DN00,
                        preferred_element_type=jnp.float32)
    z = z + lax.dot_general(Wr_ref[...], x_ref[...], _DN00,
                            preferred_element_type=jnp.float32)
    z = z + bl_ref[...]
    out_ref[...] = _elu(z)


def _tc_mid_body(s_ref, h_ref, f_ref, cnt_ref, Wl_ref, bl_ref, Wr_ref,
                 Wfu_ref, bfu_ref, g_ref, be_ref, out_ref):
    inv = 1.0 / jnp.maximum(cnt_ref[...], 1.0)
    mean = s_ref[...] * inv
    z = lax.dot_general(Wl_ref[...], mean, _DN00,
                        preferred_element_type=jnp.float32)
    z = z + lax.dot_general(Wr_ref[...], h_ref[...], _DN00,
                            preferred_element_type=jnp.float32)
    z = z + bl_ref[...]
    fused = _elu(z) + f_ref[...]
    z2 = lax.dot_general(Wfu_ref[...], fused, _DN00,
                         preferred_element_type=jnp.float32)
    z2 = z2 + bfu_ref[...]
    z2 = _ln0(z2, g_ref[...], be_ref[...])
    out_ref[...] = z2 * jax.nn.sigmoid(z2)


def _tc_final_body(s_ref, x_ref, cnt_ref, Wfl_ref, bfl_ref, Wfr_ref, out_ref):
    inv = 1.0 / jnp.maximum(cnt_ref[...], 1.0)
    mean = s_ref[...] * inv
    z = lax.dot_general(mean, Wfl_ref[...], _DN00,
                        preferred_element_type=jnp.float32)
    z = z + lax.dot_general(x_ref[...], Wfr_ref[...], _DN00,
                            preferred_element_type=jnp.float32)
    z = z + bfl_ref[...]
    out_ref[...] = _elu(z)


def _tc_call(body, n_in, out_shape):
    return pl.pallas_call(body, out_shape=out_shape)


_tc_found = pl.pallas_call(
    _tc_found_body, out_shape=jax.ShapeDtypeStruct((D, N), jnp.float32))
_tc_sage = pl.pallas_call(
    _tc_sage_body, out_shape=jax.ShapeDtypeStruct((D, N), jnp.float32))
_tc_mid = pl.pallas_call(
    _tc_mid_body, out_shape=jax.ShapeDtypeStruct((D, N), jnp.float32))
_tc_final = pl.pallas_call(
    _tc_final_body, out_shape=jax.ShapeDtypeStruct((N, D), jnp.float32))


def kernel(x_foundation, x_expression, Wl0, bl0, Wr0, Wl1, bl1, Wr1,
           Wf, bf, g1, be1, Wfu, bfu, g2, be2, Wfl, bfl, Wfr,
           ppi_edge_index):
    src = ppi_edge_index[0]
    dst = ppi_edge_index[1]
    xeT = x_expression.T

    s0f, cnt = _get_segsum(True)(xeT.reshape(-1), src, dst)
    s0T = s0f.reshape(D, N)
    cnt_r = cnt.reshape(1, N)

    fT = _tc_found(x_foundation, Wf, bf.reshape(D, 1), g1.reshape(D, 1),
                   be1.reshape(D, 1))
    h1T = _tc_sage(s0T, xeT, cnt_r, Wl0, bl0.reshape(D, 1), Wr0)

    (s1f,) = _get_segsum(False)(h1T.reshape(-1), src, dst)
    s1T = s1f.reshape(D, N)
    preT = _tc_mid(s1T, h1T, fT, cnt_r, Wl1, bl1.reshape(D, 1), Wr1,
                   Wfu, bfu.reshape(D, 1), g2.reshape(D, 1), be2.reshape(D, 1))

    (s2f,) = _get_segsum(False)(preT.reshape(-1), src, dst)
    s2T = s2f.reshape(D, N)
    out = _tc_final(s2T, preT, cnt_r, Wfl, bfl.reshape(1, D), Wfr)
    return out


# speedup vs baseline: 7.3547x; 2.2296x over previous
import functools

import jax
import jax.numpy as jnp
from jax import lax
from jax.experimental import pallas as pl
from jax.experimental.pallas import tpu as pltpu
from jax.experimental.pallas import tpu_sc as plsc

N = 10000
E = 320000
D = 128
D_FM = 512

NC = 2
NS = 16
NW = NC * NS
FPW = D // NW
CH = 6400
NCHUNK = E // CH
UNROLL = 8


def _make_segsum(with_counts: bool):
    mesh = plsc.VectorSubcoreMesh(
        core_axis_name="c", subcore_axis_name="s",
        num_cores=NC, num_subcores=NS)

    out_type = [jax.ShapeDtypeStruct((D * N,), jnp.float32)]
    scratch = [
        pltpu.VMEM((FPW * N,), jnp.float32),
        pltpu.VMEM((FPW * N,), jnp.float32),
        pltpu.VMEM((CH,), jnp.int32),
        pltpu.VMEM((CH,), jnp.int32),
        pltpu.VMEM((CH,), jnp.int32),
        pltpu.VMEM((CH,), jnp.int32),
        pltpu.SemaphoreType.DMA,
        pltpu.SemaphoreType.DMA,
        pltpu.SemaphoreType.DMA,
        pltpu.SemaphoreType.DMA,
    ]
    if with_counts:
        out_type.append(jax.ShapeDtypeStruct((N,), jnp.float32))
        scratch.append(pltpu.VMEM((N,), jnp.float32))

    def body(xT, src, dst, *rest):
        if with_counts:
            (outT, cnt_out, table_v, acc_v, sb0, sb1, db0, db1,
             ss0, ss1, ds0, ds1, cnt_v) = rest
        else:
            (outT, table_v, acc_v, sb0, sb1, db0, db1,
             ss0, ss1, ds0, ds1) = rest
            cnt_v = None
        slots = ((sb0, db0, ss0, ds0), (sb1, db1, ss1, ds1))

        def fill(c, slot):
            sb, db, ssem, dsem = slots[slot]
            off = pl.multiple_of(c * CH, 8)
            pltpu.async_copy(src.at[pl.ds(off, CH)], sb, ssem)
            pltpu.async_copy(dst.at[pl.ds(off, CH)], db, dsem)

        def drain(c, slot):
            sb, db, ssem, dsem = slots[slot]
            off = pl.multiple_of(c * CH, 8)
            pltpu.make_async_copy(src.at[pl.ds(off, CH)], sb, ssem).wait()
            pltpu.make_async_copy(dst.at[pl.ds(off, CH)], db, dsem).wait()

        fill(0, 0)
        fill(1, 1)

        wid = lax.axis_index("s") * NC + lax.axis_index("c")
        r0 = pl.multiple_of(wid * (FPW * N), 8)
        pltpu.sync_copy(xT.at[pl.ds(r0, FPW * N)], table_v)

        zeros = jnp.zeros((16,), jnp.float32)

        def zbody(j, carry):
            o = pl.multiple_of(j * 80, 16)
            for u in range(5):
                for f in range(FPW):
                    acc_v[pl.ds(o + u * 16 + f * N, 16)] = zeros
                if with_counts:
                    cnt_v[pl.ds(o + u * 16, 16)] = zeros
            return carry
        lax.fori_loop(0, N // 80, zbody, 0)

        ones = jnp.ones((16,), jnp.float32)

        def process(sb, db):
            @plsc.parallel_loop(0, CH // 16, unroll=UNROLL)
            def _(j):
                o = pl.multiple_of(j * 16, 16)
                s16 = sb[pl.ds(o, 16)]
                d16 = db[pl.ds(o, 16)]
                for f in range(FPW):
                    v = plsc.load_gather(table_v, [s16 + (f * N)])
                    plsc.addupdate_scatter(acc_v, [d16 + (f * N)], v)
                if with_counts:
                    plsc.addupdate_scatter(cnt_v, [d16], ones)

        def pair(p, carry):
            for slot in range(2):
                c = p * 2 + slot
                drain(c, slot)
                process(*slots[slot][:2])

                @pl.when(c + 2 < NCHUNK)
                def _():
                    fill(c + 2, slot)
            return carry
        lax.fori_loop(0, NCHUNK // 2, pair, 0)

        pltpu.sync_copy(acc_v, outT.at[pl.ds(r0, FPW * N)])
        if with_counts:
            @pl.when(wid == 0)
            def _():
                pltpu.sync_copy(cnt_v, cnt_out)

    return pl.kernel(body, out_type=out_type, mesh=mesh,
                     scratch_types=scratch,
                     compiler_params=pltpu.CompilerParams(
                         needs_layout_passes=False))


@functools.lru_cache(maxsize=None)
def _get_segsum(with_counts: bool):
    return _make_segsum(with_counts)


_DN00 = (((0,), (0,)), ((), ()))
_DN01 = (((0,), (1,)), ((), ()))


def _elu(z):
    return jnp.where(z > 0, z, jnp.exp(jnp.minimum(z, 0.0)) - 1.0)


def _ln0(z, g, b):
    m = jnp.mean(z, axis=0, keepdims=True)
    v = jnp.mean((z - m) * (z - m), axis=0, keepdims=True)
    return (z - m) * lax.rsqrt(v + 1e-5) * g + b


def _tc_found_body(xf_ref, Wf_ref, bf_ref, g_ref, be_ref, out_ref):
    z = lax.dot_general(Wf_ref[...], xf_ref[...], _DN01,
                        preferred_element_type=jnp.float32)
    z = z + bf_ref[...]
    z = _ln0(z, g_ref[...], be_ref[...])
    out_ref[...] = 0.5 * z * (1.0 + lax.erf(z * 0.7071067811865476))


def _tc_sage_body(s_ref, x_ref, cnt_ref, Wl_ref, bl_ref, Wr_ref, out_ref):
    inv = 1.0 / jnp.maximum(cnt_ref[...], 1.0)
    mean = s_ref[...] * inv
    z = lax.dot_general(Wl_ref[...], mean, _DN00,
                        preferred_element_type=jnp.float32)
    z = z + lax.dot_general(Wr_ref[...], x_ref[...], _DN00,
                            preferred_element_type=jnp.float32)
    z = z + bl_ref[...]
    out_ref[...] = _elu(z)


def _tc_mid_body(s_ref, h_ref, f_ref, cnt_ref, Wl_ref, bl_ref, Wr_ref,
                 Wfu_ref, bfu_ref, g_ref, be_ref, out_ref):
    inv = 1.0 / jnp.maximum(cnt_ref[...], 1.0)
    mean = s_ref[...] * inv
    z = lax.dot_general(Wl_ref[...], mean, _DN00,
                        preferred_element_type=jnp.float32)
    z = z + lax.dot_general(Wr_ref[...], h_ref[...], _DN00,
                            preferred_element_type=jnp.float32)
    z = z + bl_ref[...]
    fused = _elu(z) + f_ref[...]
    z2 = lax.dot_general(Wfu_ref[...], fused, _DN00,
                         preferred_element_type=jnp.float32)
    z2 = z2 + bfu_ref[...]
    z2 = _ln0(z2, g_ref[...], be_ref[...])
    out_ref[...] = z2 * jax.nn.sigmoid(z2)


def _tc_final_body(s_ref, x_ref, cnt_ref, Wfl_ref, bfl_ref, Wfr_ref, out_ref):
    inv = 1.0 / jnp.maximum(cnt_ref[...], 1.0)
    mean = s_ref[...] * inv
    z = lax.dot_general(mean, Wfl_ref[...], _DN00,
                        preferred_element_type=jnp.float32)
    z = z + lax.dot_general(x_ref[...], Wfr_ref[...], _DN00,
                            preferred_element_type=jnp.float32)
    z = z + bfl_ref[...]
    out_ref[...] = _elu(z)


def _tc_call(body, n_in, out_shape):
    return pl.pallas_call(body, out_shape=out_shape)


_tc_found = pl.pallas_call(
    _tc_found_body, out_shape=jax.ShapeDtypeStruct((D, N), jnp.float32))
_tc_sage = pl.pallas_call(
    _tc_sage_body, out_shape=jax.ShapeDtypeStruct((D, N), jnp.float32))
_tc_mid = pl.pallas_call(
    _tc_mid_body, out_shape=jax.ShapeDtypeStruct((D, N), jnp.float32))
_tc_final = pl.pallas_call(
    _tc_final_body, out_shape=jax.ShapeDtypeStruct((N, D), jnp.float32))


def kernel(x_foundation, x_expression, Wl0, bl0, Wr0, Wl1, bl1, Wr1,
           Wf, bf, g1, be1, Wfu, bfu, g2, be2, Wfl, bfl, Wfr,
           ppi_edge_index):
    src = ppi_edge_index[0]
    dst = ppi_edge_index[1]
    xeT = x_expression.T

    s0f, cnt = _get_segsum(True)(xeT.reshape(-1), src, dst)
    s0T = s0f.reshape(D, N)
    cnt_r = cnt.reshape(1, N)

    fT = _tc_found(x_foundation, Wf, bf.reshape(D, 1), g1.reshape(D, 1),
                   be1.reshape(D, 1))
    h1T = _tc_sage(s0T, xeT, cnt_r, Wl0, bl0.reshape(D, 1), Wr0)

    (s1f,) = _get_segsum(False)(h1T.reshape(-1), src, dst)
    s1T = s1f.reshape(D, N)
    preT = _tc_mid(s1T, h1T, fT, cnt_r, Wl1, bl1.reshape(D, 1), Wr1,
                   Wfu, bfu.reshape(D, 1), g2.reshape(D, 1), be2.reshape(D, 1))

    (s2f,) = _get_segsum(False)(preT.reshape(-1), src, dst)
    s2T = s2f.reshape(D, N)
    out = _tc_final(s2T, preT, cnt_r, Wfl, bfl.reshape(1, D), Wfr)
    return out
